# Initial kernel scaffold; baseline (speedup 1.0000x reference)
import jax
import jax.numpy as jnp
from jax.experimental import pallas as pl


def kernel(x, edge_index, W1l, b1l, W1r, W2l, b2l, W2r):
    raise NotImplementedError("write your pallas kernel here")


# speedup vs baseline: 2.9998x; 2.9998x over previous
import functools

import jax
import jax.numpy as jnp
from jax import lax
from jax.experimental import pallas as pl
from jax.experimental.pallas import tpu as pltpu
from jax.experimental.pallas import tpu_sc as plsc

N = 10000
E = 320000
D = 128

NUM_CORES = 2
NUM_SUBCORES = 16
NUM_TILES = NUM_CORES * NUM_SUBCORES

CHUNK = 128
EPW = 10240
E_PAD = EPW * NUM_TILES
N_PAD = 10240
ROWS_PER_TILE = N_PAD // NUM_SUBCORES
ROW_BLOCKS = ROWS_PER_TILE // CHUNK

_sc_mesh = plsc.VectorSubcoreMesh(core_axis_name="c", subcore_axis_name="s")


@functools.partial(
    pl.kernel,
    out_type=jax.ShapeDtypeStruct((NUM_CORES, N_PAD, D), jnp.float32),
    scratch_types=[
        pltpu.VMEM_SHARED((N_PAD, D), jnp.float32),
        pltpu.VMEM((CHUNK,), jnp.int32),
        pltpu.VMEM((CHUNK,), jnp.int32),
        pltpu.VMEM((CHUNK, D), jnp.float32),
        pltpu.SemaphoreType.DMA,
    ],
    mesh=_sc_mesh,
)
def _sc_seg_sum(x_hbm, src_hbm, dst_hbm, z_hbm, acc_out,
                acc_sh, sidx_v, didx_v, rows_v, sem):
    c = lax.axis_index("c")
    s = lax.axis_index("s")
    wid = c * NUM_SUBCORES + s
    r0 = s * ROWS_PER_TILE
    ew0 = wid * EPW

    pltpu.sync_copy(z_hbm, rows_v)
    for b in range(ROW_BLOCKS):
        pltpu.sync_copy(rows_v, acc_sh.at[pl.ds(r0 + b * CHUNK, CHUNK)])
    plsc.subcore_barrier()

    def chunk_body(j, carry):
        base = ew0 + j * CHUNK
        pltpu.sync_copy(src_hbm.at[pl.ds(base, CHUNK)], sidx_v)
        pltpu.sync_copy(dst_hbm.at[pl.ds(base, CHUNK)], didx_v)
        pltpu.async_copy(x_hbm.at[sidx_v], rows_v, sem).wait()
        pltpu.sync_copy(rows_v, acc_sh.at[didx_v], add=True)
        return carry

    lax.fori_loop(0, EPW // CHUNK, chunk_body, 0)
    plsc.subcore_barrier()

    for b in range(ROW_BLOCKS):
        r = r0 + b * CHUNK
        pltpu.sync_copy(acc_sh.at[pl.ds(r, CHUNK)], rows_v)
        pltpu.sync_copy(rows_v, acc_out.at[c, pl.ds(r, CHUNK)])


@functools.partial(
    pl.kernel,
    out_type=jax.ShapeDtypeStruct((NUM_CORES, N_PAD, D), jnp.float32),
    scratch_types=[
        pltpu.VMEM_SHARED((N_PAD, D), jnp.float32),
        pltpu.VMEM((CHUNK,), jnp.int32),
        pltpu.VMEM((CHUNK, D), jnp.float32),
    ],
    mesh=_sc_mesh,
)
def _sc_count(dst_hbm, z_hbm, o_hbm, cnt_out, cnt_sh, didx_v, rows_v):
    c = lax.axis_index("c")
    s = lax.axis_index("s")
    wid = c * NUM_SUBCORES + s
    r0 = s * ROWS_PER_TILE
    ew0 = wid * EPW

    pltpu.sync_copy(z_hbm, rows_v)
    for b in range(ROW_BLOCKS):
        pltpu.sync_copy(rows_v, cnt_sh.at[pl.ds(r0 + b * CHUNK, CHUNK)])
    pltpu.sync_copy(o_hbm, rows_v)
    plsc.subcore_barrier()

    def chunk_body(j, carry):
        base = ew0 + j * CHUNK
        pltpu.sync_copy(dst_hbm.at[pl.ds(base, CHUNK)], didx_v)
        pltpu.sync_copy(rows_v, cnt_sh.at[didx_v], add=True)
        return carry

    lax.fori_loop(0, EPW // CHUNK, chunk_body, 0)
    plsc.subcore_barrier()

    for b in range(ROW_BLOCKS):
        r = r0 + b * CHUNK
        pltpu.sync_copy(cnt_sh.at[pl.ds(r, CHUNK)], rows_v)
        pltpu.sync_copy(rows_v, cnt_out.at[c, pl.ds(r, CHUNK)])


def _make_tc_layer(final: bool):
    BN = 1024

    def body(acc_ref, cnt_ref, x_ref, wl_ref, bl_ref, wr_ref, o_ref):
        a = acc_ref[0] + acc_ref[1]
        cnt = cnt_ref[0, :, 0:1] + cnt_ref[1, :, 0:1]
        agg = a * (1.0 / jnp.maximum(cnt, 1.0))
        hl = lax.dot_general(agg, wl_ref[...], (((1,), (1,)), ((), ())),
                             preferred_element_type=jnp.float32)
        hr = lax.dot_general(x_ref[...], wr_ref[...], (((1,), (1,)), ((), ())),
                             preferred_element_type=jnp.float32)
        o = hl + hr + bl_ref[...]
        if final:
            m = jnp.max(o, axis=1, keepdims=True)
            lse = jnp.log(jnp.sum(jnp.exp(o - m), axis=1, keepdims=True))
            o_ref[...] = o - m - lse
        else:
            o_ref[...] = jnp.tanh(o)

    grid = (N_PAD // BN,)
    return pl.pallas_call(
        body,
        grid=grid,
        in_specs=[
            pl.BlockSpec((NUM_CORES, BN, D), lambda i: (0, i, 0)),
            pl.BlockSpec((NUM_CORES, BN, D), lambda i: (0, i, 0)),
            pl.BlockSpec((BN, D), lambda i: (i, 0)),
            pl.BlockSpec((D, D), lambda i: (0, 0)),
            pl.BlockSpec((1, D), lambda i: (0, 0)),
            pl.BlockSpec((D, D), lambda i: (0, 0)),
        ],
        out_specs=pl.BlockSpec((BN, D), lambda i: (i, 0)),
        out_shape=jax.ShapeDtypeStruct((N_PAD, D), jnp.float32),
    )


_tc_hidden = _make_tc_layer(False)
_tc_final = _make_tc_layer(True)


def kernel(x, edge_index, W1l, b1l, W1r, W2l, b2l, W2r):
    src = edge_index[0]
    dst = edge_index[1]
    pad_e = E_PAD - E
    src_p = jnp.concatenate([src, jnp.zeros((pad_e,), jnp.int32)])
    dst_p = jnp.concatenate([dst, jnp.full((pad_e,), N, jnp.int32)])
    x_p = jnp.pad(x, ((0, N_PAD - N), (0, 0)))

    z = jnp.zeros((CHUNK, D), jnp.float32)
    o128 = jnp.ones((CHUNK, D), jnp.float32)

    cnt = _sc_count(dst_p, z, o128)
    acc1 = _sc_seg_sum(x_p, src_p, dst_p, z)
    h = _tc_hidden(acc1, cnt, x_p, W1l, b1l.reshape(1, D), W1r)
    acc2 = _sc_seg_sum(h, src_p, dst_p, z)
    o = _tc_final(acc2, cnt, h, W2l, b2l.reshape(1, D), W2r)
    return o[:N]


# speedup vs baseline: 3.8281x; 1.2761x over previous
import functools

import jax
import jax.numpy as jnp
from jax import lax
from jax.experimental import pallas as pl
from jax.experimental.pallas import tpu as pltpu
from jax.experimental.pallas import tpu_sc as plsc

N = 10000
E = 320000
D = 128

NUM_CORES = 2
NUM_SUBCORES = 16
NUM_TILES = NUM_CORES * NUM_SUBCORES

CHUNK = 64
EPW = 10240
E_PAD = EPW * NUM_TILES
N_PAD = 10240
ROWS_PER_TILE = N_PAD // NUM_SUBCORES
ROW_BLOCKS = ROWS_PER_TILE // CHUNK
NCH = EPW // CHUNK
NGRP = NCH // 8

_sc_mesh = plsc.VectorSubcoreMesh(core_axis_name="c", subcore_axis_name="s")


@functools.partial(
    pl.kernel,
    out_type=jax.ShapeDtypeStruct((NUM_CORES, N_PAD, D), jnp.float32),
    scratch_types=[
        pltpu.VMEM_SHARED((N_PAD, D), jnp.float32),
        pltpu.VMEM((8, CHUNK), jnp.int32),
        pltpu.VMEM((8, CHUNK), jnp.int32),
        pltpu.VMEM((CHUNK, D), jnp.float32),
        pltpu.VMEM((CHUNK, D), jnp.float32),
        pltpu.VMEM((CHUNK, D), jnp.float32),
        pltpu.VMEM((CHUNK, D), jnp.float32),
        pltpu.SemaphoreType.DMA,
        pltpu.SemaphoreType.DMA,
        pltpu.SemaphoreType.DMA,
        pltpu.SemaphoreType.DMA,
        pltpu.SemaphoreType.DMA,
        pltpu.SemaphoreType.DMA,
        pltpu.SemaphoreType.DMA,
        pltpu.SemaphoreType.DMA,
        pltpu.SemaphoreType.DMA,
        pltpu.SemaphoreType.DMA,
        pltpu.SemaphoreType.DMA,
        pltpu.SemaphoreType.DMA,
        pltpu.SemaphoreType.DMA,
        pltpu.SemaphoreType.DMA,
        pltpu.SemaphoreType.DMA,
        pltpu.SemaphoreType.DMA,
    ],
    mesh=_sc_mesh,
)
def _sc_seg_sum(x_hbm, src_hbm, dst_hbm, z_hbm, acc_out, acc_sh,
                sidx8, didx8, rv0, rv1, rv2, rv3, *sems):
    rows = [rv0, rv1, rv2, rv3]
    gsem = list(sems[0:4])
    ssem = list(sems[4:8])
    isem = list(sems[8:16])

    c = lax.axis_index("c")
    s = lax.axis_index("s")
    wid = c * NUM_SUBCORES + s
    r0 = s * ROWS_PER_TILE
    w0 = wid * NCH

    def idx_copies(jj, k):
        pltpu.async_copy(src_hbm.at[w0 + jj], sidx8.at[k], isem[k])
        pltpu.async_copy(dst_hbm.at[w0 + jj], didx8.at[k], isem[k])

    def idx_wait(jj, k):
        pltpu.make_async_copy(src_hbm.at[w0 + jj], sidx8.at[k], isem[k]).wait()
        pltpu.make_async_copy(dst_hbm.at[w0 + jj], didx8.at[k], isem[k]).wait()

    def gather_start(k8, k4):
        pltpu.async_copy(x_hbm.at[sidx8.at[k8]], rows[k4], gsem[k4])

    def gather_wait(k8, k4):
        pltpu.make_async_copy(
            x_hbm.at[sidx8.at[k8]], rows[k4], gsem[k4]).wait()

    def scatter_start(k8, k4):
        pltpu.async_copy(rows[k4], acc_sh.at[didx8.at[k8]], ssem[k4], add=True)

    def scatter_wait(k8, k4):
        pltpu.make_async_copy(
            rows[k4], acc_sh.at[didx8.at[k8]], ssem[k4]).wait()

    pltpu.sync_copy(src_hbm.at[w0], sidx8.at[0])
    pltpu.sync_copy(dst_hbm.at[w0], didx8.at[0])
    pltpu.sync_copy(src_hbm.at[w0 + 1], sidx8.at[1])
    pltpu.sync_copy(dst_hbm.at[w0 + 1], didx8.at[1])
    idx_copies(2, 2)
    idx_copies(3, 3)

    pltpu.sync_copy(z_hbm, rows[0])
    for b in range(ROW_BLOCKS):
        pltpu.sync_copy(rows[0], acc_sh.at[pl.ds(r0 + b * CHUNK, CHUNK)])
    plsc.subcore_barrier()

    gather_start(0, 0)
    gather_start(1, 1)

    def group_body(g, carry):
        for k in range(8):
            j = 8 * g + k
            k4 = k % 4
            gather_wait(k, k4)
            scatter_start(k, k4)

            if k < 4:
                idx_copies(j + 4, (k + 4) % 8)
            else:
                @pl.when(g < NGRP - 1)
                def _():
                    idx_copies(j + 4, (k + 4) % 8)

            if k < 2:
                @pl.when(g >= 1)
                def _():
                    scatter_wait((k - 2) % 8, (k - 2) % 4)
            else:
                scatter_wait(k - 2, (k - 2) % 4)

            if k < 6:
                idx_wait(j + 2, (k + 2) % 8)
                gather_start((k + 2) % 8, (k + 2) % 4)
            else:
                @pl.when(g < NGRP - 1)
                def _():
                    idx_wait(j + 2, (k + 2) % 8)
                    gather_start((k + 2) % 8, (k + 2) % 4)
        return carry

    lax.fori_loop(0, NGRP, group_body, 0)

    scatter_wait(6, 2)
    scatter_wait(7, 3)
    plsc.subcore_barrier()

    for b in range(ROW_BLOCKS):
        r = r0 + b * CHUNK
        pltpu.sync_copy(acc_sh.at[pl.ds(r, CHUNK)], rows[0])
        pltpu.sync_copy(rows[0], acc_out.at[c, pl.ds(r, CHUNK)])


CNT_BATCH = 8


@functools.partial(
    pl.kernel,
    out_type=jax.ShapeDtypeStruct((NUM_CORES, N_PAD, D), jnp.float32),
    scratch_types=[
        pltpu.VMEM_SHARED((N_PAD, D), jnp.float32),
        pltpu.VMEM((NCH, CHUNK), jnp.int32),
        pltpu.VMEM((CHUNK, D), jnp.float32),
        pltpu.SemaphoreType.DMA,
    ],
    mesh=_sc_mesh,
)
def _sc_count(dst_hbm, z_hbm, o_hbm, cnt_out, cnt_sh, didx_all, stage_v, csem):
    c = lax.axis_index("c")
    s = lax.axis_index("s")
    wid = c * NUM_SUBCORES + s
    r0 = s * ROWS_PER_TILE
    w0 = wid * NCH

    pltpu.sync_copy(dst_hbm.at[pl.ds(w0, NCH)], didx_all)
    pltpu.sync_copy(z_hbm, stage_v)
    for b in range(ROW_BLOCKS):
        pltpu.sync_copy(stage_v, cnt_sh.at[pl.ds(r0 + b * CHUNK, CHUNK)])
    pltpu.sync_copy(o_hbm, stage_v)
    plsc.subcore_barrier()

    def batch_body(g, carry):
        for b in range(CNT_BATCH):
            pltpu.async_copy(
                stage_v, cnt_sh.at[didx_all.at[g * CNT_BATCH + b]], csem,
                add=True)
        for b in range(CNT_BATCH):
            pltpu.make_async_copy(
                stage_v, cnt_sh.at[didx_all.at[g * CNT_BATCH + b]],
                csem).wait()
        return carry

    lax.fori_loop(0, NCH // CNT_BATCH, batch_body, 0)
    plsc.subcore_barrier()

    for b in range(ROW_BLOCKS):
        r = r0 + b * CHUNK
        pltpu.sync_copy(cnt_sh.at[pl.ds(r, CHUNK)], stage_v)
        pltpu.sync_copy(stage_v, cnt_out.at[c, pl.ds(r, CHUNK)])


def _make_tc_layer(final: bool):
    BN = 1024

    def body(acc_ref, cnt_ref, x_ref, wl_ref, bl_ref, wr_ref, o_ref):
        a = acc_ref[0] + acc_ref[1]
        cnt = cnt_ref[0, :, 0:1] + cnt_ref[1, :, 0:1]
        agg = a * (1.0 / jnp.maximum(cnt, 1.0))
        hl = lax.dot_general(agg, wl_ref[...], (((1,), (1,)), ((), ())),
                             preferred_element_type=jnp.float32)
        hr = lax.dot_general(x_ref[...], wr_ref[...], (((1,), (1,)), ((), ())),
                             preferred_element_type=jnp.float32)
        o = hl + hr + bl_ref[...]
        if final:
            m = jnp.max(o, axis=1, keepdims=True)
            lse = jnp.log(jnp.sum(jnp.exp(o - m), axis=1, keepdims=True))
            o_ref[...] = o - m - lse
        else:
            o_ref[...] = jnp.tanh(o)

    grid = (N_PAD // BN,)
    return pl.pallas_call(
        body,
        grid=grid,
        in_specs=[
            pl.BlockSpec((NUM_CORES, BN, D), lambda i: (0, i, 0)),
            pl.BlockSpec((NUM_CORES, BN, D), lambda i: (0, i, 0)),
            pl.BlockSpec((BN, D), lambda i: (i, 0)),
            pl.BlockSpec((D, D), lambda i: (0, 0)),
            pl.BlockSpec((1, D), lambda i: (0, 0)),
            pl.BlockSpec((D, D), lambda i: (0, 0)),
        ],
        out_specs=pl.BlockSpec((BN, D), lambda i: (i, 0)),
        out_shape=jax.ShapeDtypeStruct((N_PAD, D), jnp.float32),
    )


_tc_hidden = _make_tc_layer(False)
_tc_final = _make_tc_layer(True)


def kernel(x, edge_index, W1l, b1l, W1r, W2l, b2l, W2r):
    src = edge_index[0]
    dst = edge_index[1]
    pad_e = E_PAD - E
    src_p = jnp.concatenate([src, jnp.zeros((pad_e,), jnp.int32)])
    dst_p = jnp.concatenate([dst, jnp.full((pad_e,), N, jnp.int32)])
    src2 = src_p.reshape(E_PAD // CHUNK, CHUNK)
    dst2 = dst_p.reshape(E_PAD // CHUNK, CHUNK)
    x_p = jnp.pad(x, ((0, N_PAD - N), (0, 0)))

    z = jnp.zeros((CHUNK, D), jnp.float32)
    o128 = jnp.ones((CHUNK, D), jnp.float32)

    cnt = _sc_count(dst2, z, o128)
    acc1 = _sc_seg_sum(x_p, src2, dst2, z)
    h = _tc_hidden(acc1, cnt, x_p, W1l, b1l.reshape(1, D), W1r)
    acc2 = _sc_seg_sum(h, src2, dst2, z)
    o = _tc_final(acc2, cnt, h, W2l, b2l.reshape(1, D), W2r)
    return o[:N]


# speedup vs baseline: 9.7964x; 2.5591x over previous
import functools

import jax
import jax.numpy as jnp
from jax import lax
from jax.experimental import pallas as pl
from jax.experimental.pallas import tpu as pltpu
from jax.experimental.pallas import tpu_sc as plsc

N = 10000
E = 320000
D = 128

NUM_CORES = 2
NUM_SUBCORES = 16
NUM_TILES = NUM_CORES * NUM_SUBCORES

CHUNK = 64
EPW = 10240
E_PAD = EPW * NUM_TILES
N_PAD = 10240
ROWS_PER_TILE = N_PAD // NUM_SUBCORES
ROW_BLOCKS = ROWS_PER_TILE // CHUNK
NCH = EPW // CHUNK
NGRP = NCH // 8

_sc_mesh = plsc.VectorSubcoreMesh(core_axis_name="c", subcore_axis_name="s")


@functools.partial(
    pl.kernel,
    out_type=jax.ShapeDtypeStruct((NUM_CORES, N_PAD, D), jnp.float32),
    scratch_types=[
        pltpu.VMEM_SHARED((N_PAD, D), jnp.float32),
        pltpu.VMEM((8, CHUNK), jnp.int32),
        pltpu.VMEM((8, CHUNK), jnp.int32),
        pltpu.VMEM((CHUNK, D), jnp.float32),
        pltpu.VMEM((CHUNK, D), jnp.float32),
        pltpu.VMEM((CHUNK, D), jnp.float32),
        pltpu.VMEM((CHUNK, D), jnp.float32),
        pltpu.SemaphoreType.DMA,
        pltpu.SemaphoreType.DMA,
        pltpu.SemaphoreType.DMA,
        pltpu.SemaphoreType.DMA,
        pltpu.SemaphoreType.DMA,
        pltpu.SemaphoreType.DMA,
        pltpu.SemaphoreType.DMA,
        pltpu.SemaphoreType.DMA,
        pltpu.SemaphoreType.DMA,
        pltpu.SemaphoreType.DMA,
        pltpu.SemaphoreType.DMA,
        pltpu.SemaphoreType.DMA,
        pltpu.SemaphoreType.DMA,
        pltpu.SemaphoreType.DMA,
        pltpu.SemaphoreType.DMA,
        pltpu.SemaphoreType.DMA,
    ],
    mesh=_sc_mesh,
)
def _sc_seg_sum(x_hbm, src_hbm, dst_hbm, z_hbm, acc_out, acc_sh,
                sidx8, didx8, rv0, rv1, rv2, rv3, *sems):
    rows = [rv0, rv1, rv2, rv3]
    gsem = list(sems[0:4])
    ssem = list(sems[4:8])
    isem = list(sems[8:16])

    c = lax.axis_index("c")
    s = lax.axis_index("s")
    wid = c * NUM_SUBCORES + s
    r0 = s * ROWS_PER_TILE
    w0 = wid * NCH

    def idx_copies(jj, k):
        pltpu.async_copy(src_hbm.at[w0 + jj], sidx8.at[k], isem[k])
        pltpu.async_copy(dst_hbm.at[w0 + jj], didx8.at[k], isem[k])

    def idx_wait(jj, k):
        pltpu.make_async_copy(src_hbm.at[w0 + jj], sidx8.at[k], isem[k]).wait()
        pltpu.make_async_copy(dst_hbm.at[w0 + jj], didx8.at[k], isem[k]).wait()

    def gather_start(k8, k4):
        pltpu.async_copy(x_hbm.at[sidx8.at[k8]], rows[k4], gsem[k4])

    def gather_wait(k8, k4):
        pltpu.make_async_copy(
            x_hbm.at[sidx8.at[k8]], rows[k4], gsem[k4]).wait()

    def scatter_start(k8, k4):
        pltpu.async_copy(rows[k4], acc_sh.at[didx8.at[k8]], ssem[k4], add=True)

    def scatter_wait(k8, k4):
        pltpu.make_async_copy(
            rows[k4], acc_sh.at[didx8.at[k8]], ssem[k4]).wait()

    pltpu.sync_copy(src_hbm.at[w0], sidx8.at[0])
    pltpu.sync_copy(dst_hbm.at[w0], didx8.at[0])
    pltpu.sync_copy(src_hbm.at[w0 + 1], sidx8.at[1])
    pltpu.sync_copy(dst_hbm.at[w0 + 1], didx8.at[1])
    idx_copies(2, 2)
    idx_copies(3, 3)

    pltpu.sync_copy(z_hbm, rows[0])
    for b in range(ROW_BLOCKS):
        pltpu.sync_copy(rows[0], acc_sh.at[pl.ds(r0 + b * CHUNK, CHUNK)])
    plsc.subcore_barrier()

    gather_start(0, 0)
    gather_start(1, 1)

    def group_body(g, carry):
        for k in range(8):
            j = 8 * g + k
            k4 = k % 4
            gather_wait(k, k4)
            scatter_start(k, k4)

            if k < 4:
                idx_copies(j + 4, (k + 4) % 8)
            else:
                @pl.when(g < NGRP - 1)
                def _():
                    idx_copies(j + 4, (k + 4) % 8)

            if k < 2:
                @pl.when(g >= 1)
                def _():
                    scatter_wait((k - 2) % 8, (k - 2) % 4)
            else:
                scatter_wait(k - 2, (k - 2) % 4)

            if k < 6:
                idx_wait(j + 2, (k + 2) % 8)
                gather_start((k + 2) % 8, (k + 2) % 4)
            else:
                @pl.when(g < NGRP - 1)
                def _():
                    idx_wait(j + 2, (k + 2) % 8)
                    gather_start((k + 2) % 8, (k + 2) % 4)
        return carry

    lax.fori_loop(0, NGRP, group_body, 0)

    scatter_wait(6, 2)
    scatter_wait(7, 3)
    plsc.subcore_barrier()

    for b in range(ROW_BLOCKS):
        r = r0 + b * CHUNK
        pltpu.sync_copy(acc_sh.at[pl.ds(r, CHUNK)], rows[0])
        pltpu.sync_copy(rows[0], acc_out.at[c, pl.ds(r, CHUNK)])


CNT_BATCH = 8


@functools.partial(
    pl.kernel,
    out_type=jax.ShapeDtypeStruct((NUM_CORES, N_PAD, D), jnp.float32),
    scratch_types=[
        pltpu.VMEM_SHARED((N_PAD, D), jnp.float32),
        pltpu.VMEM((NCH, CHUNK), jnp.int32),
        pltpu.VMEM((CHUNK, D), jnp.float32),
        pltpu.SemaphoreType.DMA,
    ],
    mesh=_sc_mesh,
)
def _sc_count(dst_hbm, z_hbm, o_hbm, cnt_out, cnt_sh, didx_all, stage_v, csem):
    c = lax.axis_index("c")
    s = lax.axis_index("s")
    wid = c * NUM_SUBCORES + s
    r0 = s * ROWS_PER_TILE
    w0 = wid * NCH

    pltpu.sync_copy(dst_hbm.at[pl.ds(w0, NCH)], didx_all)
    pltpu.sync_copy(z_hbm, stage_v)
    for b in range(ROW_BLOCKS):
        pltpu.sync_copy(stage_v, cnt_sh.at[pl.ds(r0 + b * CHUNK, CHUNK)])
    pltpu.sync_copy(o_hbm, stage_v)
    plsc.subcore_barrier()

    def batch_body(g, carry):
        for b in range(CNT_BATCH):
            pltpu.async_copy(
                stage_v, cnt_sh.at[didx_all.at[g * CNT_BATCH + b]], csem,
                add=True)
        for b in range(CNT_BATCH):
            pltpu.make_async_copy(
                stage_v, cnt_sh.at[didx_all.at[g * CNT_BATCH + b]],
                csem).wait()
        return carry

    lax.fori_loop(0, NCH // CNT_BATCH, batch_body, 0)
    plsc.subcore_barrier()

    for b in range(ROW_BLOCKS):
        r = r0 + b * CHUNK
        pltpu.sync_copy(cnt_sh.at[pl.ds(r, CHUNK)], stage_v)
        pltpu.sync_copy(stage_v, cnt_out.at[c, pl.ds(r, CHUNK)])


def _make_tc_layer(final: bool):
    BN = 1024

    def body(acc_ref, cnt_ref, x_ref, wl_ref, bl_ref, wr_ref, o_ref):
        a = acc_ref[0] + acc_ref[1]
        cnt = cnt_ref[0, :, 0:1] + cnt_ref[1, :, 0:1]
        agg = a * (1.0 / jnp.maximum(cnt, 1.0))
        hl = lax.dot_general(agg, wl_ref[...], (((1,), (1,)), ((), ())),
                             preferred_element_type=jnp.float32)
        hr = lax.dot_general(x_ref[...], wr_ref[...], (((1,), (1,)), ((), ())),
                             preferred_element_type=jnp.float32)
        o = hl + hr + bl_ref[...]
        if final:
            m = jnp.max(o, axis=1, keepdims=True)
            lse = jnp.log(jnp.sum(jnp.exp(o - m), axis=1, keepdims=True))
            o_ref[...] = o - m - lse
        else:
            o_ref[...] = jnp.tanh(o)

    grid = (N_PAD // BN,)
    return pl.pallas_call(
        body,
        grid=grid,
        in_specs=[
            pl.BlockSpec((NUM_CORES, BN, D), lambda i: (0, i, 0)),
            pl.BlockSpec((NUM_CORES, BN, D), lambda i: (0, i, 0)),
            pl.BlockSpec((BN, D), lambda i: (i, 0)),
            pl.BlockSpec((D, D), lambda i: (0, 0)),
            pl.BlockSpec((1, D), lambda i: (0, 0)),
            pl.BlockSpec((D, D), lambda i: (0, 0)),
        ],
        out_specs=pl.BlockSpec((BN, D), lambda i: (i, 0)),
        out_shape=jax.ShapeDtypeStruct((N_PAD, D), jnp.float32),
    )


_tc_hidden = _make_tc_layer(False)
_tc_final = _make_tc_layer(True)


def kernel(x, edge_index, W1l, b1l, W1r, W2l, b2l, W2r):
    src = edge_index[0]
    dst = edge_index[1]
    pad_e = E_PAD - E
    pad_i = jnp.arange(pad_e, dtype=jnp.int32)
    src_p = jnp.concatenate([src, pad_i % N])
    dst_p = jnp.concatenate([dst, N + pad_i % (N_PAD - N)])
    src2 = src_p.reshape(E_PAD // CHUNK, CHUNK)
    dst2 = dst_p.reshape(E_PAD // CHUNK, CHUNK)
    x_p = jnp.pad(x, ((0, N_PAD - N), (0, 0)))

    z = jnp.zeros((CHUNK, D), jnp.float32)
    o128 = jnp.ones((CHUNK, D), jnp.float32)

    cnt = _sc_count(dst2, z, o128)
    acc1 = _sc_seg_sum(x_p, src2, dst2, z)
    h = _tc_hidden(acc1, cnt, x_p, W1l, b1l.reshape(1, D), W1r)
    acc2 = _sc_seg_sum(h, src2, dst2, z)
    o = _tc_final(acc2, cnt, h, W2l, b2l.reshape(1, D), W2r)
    return o[:N]


# speedup vs baseline: 9.8712x; 1.0076x over previous
import functools

import jax
import jax.numpy as jnp
from jax import lax
from jax.experimental import pallas as pl
from jax.experimental.pallas import tpu as pltpu
from jax.experimental.pallas import tpu_sc as plsc

N = 10000
E = 320000
D = 128

NUM_CORES = 2
NUM_SUBCORES = 16
NUM_TILES = NUM_CORES * NUM_SUBCORES

CHUNK = 64
EPW = 10240
E_PAD = EPW * NUM_TILES
N_PAD = 10240
ROWS_PER_TILE = N_PAD // NUM_SUBCORES
ROW_BLOCKS = ROWS_PER_TILE // CHUNK
NCH = EPW // CHUNK
NGRP = NCH // 8

_sc_mesh = plsc.VectorSubcoreMesh(core_axis_name="c", subcore_axis_name="s")


@functools.partial(
    pl.kernel,
    out_type=jax.ShapeDtypeStruct((NUM_CORES, N_PAD, D), jnp.float32),
    scratch_types=[
        pltpu.VMEM_SHARED((N_PAD, D), jnp.float32),
        pltpu.VMEM((8, CHUNK), jnp.int32),
        pltpu.VMEM((8, CHUNK), jnp.int32),
        pltpu.VMEM((CHUNK, D), jnp.float32),
        pltpu.VMEM((CHUNK, D), jnp.float32),
        pltpu.VMEM((CHUNK, D), jnp.float32),
        pltpu.VMEM((CHUNK, D), jnp.float32),
        pltpu.SemaphoreType.DMA,
        pltpu.SemaphoreType.DMA,
        pltpu.SemaphoreType.DMA,
        pltpu.SemaphoreType.DMA,
        pltpu.SemaphoreType.DMA,
        pltpu.SemaphoreType.DMA,
        pltpu.SemaphoreType.DMA,
        pltpu.SemaphoreType.DMA,
        pltpu.SemaphoreType.DMA,
        pltpu.SemaphoreType.DMA,
        pltpu.SemaphoreType.DMA,
        pltpu.SemaphoreType.DMA,
        pltpu.SemaphoreType.DMA,
        pltpu.SemaphoreType.DMA,
        pltpu.SemaphoreType.DMA,
        pltpu.SemaphoreType.DMA,
    ],
    mesh=_sc_mesh,
)
def _sc_seg_sum(x_hbm, src_hbm, dst_hbm, z_hbm, acc_out, acc_sh,
                sidx8, didx8, rv0, rv1, rv2, rv3, *sems):
    rows = [rv0, rv1, rv2, rv3]
    gsem = list(sems[0:4])
    ssem = list(sems[4:8])
    isem = list(sems[8:16])

    c = lax.axis_index("c")
    s = lax.axis_index("s")
    wid = c * NUM_SUBCORES + s
    r0 = s * ROWS_PER_TILE
    w0 = wid * NCH

    def idx_copies(jj, k):
        pltpu.async_copy(src_hbm.at[w0 + jj], sidx8.at[k], isem[k])
        pltpu.async_copy(dst_hbm.at[w0 + jj], didx8.at[k], isem[k])

    def idx_wait(jj, k):
        pltpu.make_async_copy(src_hbm.at[w0 + jj], sidx8.at[k], isem[k]).wait()
        pltpu.make_async_copy(dst_hbm.at[w0 + jj], didx8.at[k], isem[k]).wait()

    def gather_start(k8, k4):
        pltpu.async_copy(x_hbm.at[sidx8.at[k8]], rows[k4], gsem[k4])

    def gather_wait(k8, k4):
        pltpu.make_async_copy(
            x_hbm.at[sidx8.at[k8]], rows[k4], gsem[k4]).wait()

    def scatter_start(k8, k4):
        pltpu.async_copy(rows[k4], acc_sh.at[didx8.at[k8]], ssem[k4], add=True)

    def scatter_wait(k8, k4):
        pltpu.make_async_copy(
            rows[k4], acc_sh.at[didx8.at[k8]], ssem[k4]).wait()

    pltpu.sync_copy(src_hbm.at[w0], sidx8.at[0])
    pltpu.sync_copy(dst_hbm.at[w0], didx8.at[0])
    pltpu.sync_copy(src_hbm.at[w0 + 1], sidx8.at[1])
    pltpu.sync_copy(dst_hbm.at[w0 + 1], didx8.at[1])
    idx_copies(2, 2)
    idx_copies(3, 3)

    pltpu.sync_copy(z_hbm, rows[0])
    for b in range(ROW_BLOCKS):
        pltpu.sync_copy(rows[0], acc_sh.at[pl.ds(r0 + b * CHUNK, CHUNK)])
    plsc.subcore_barrier()

    gather_start(0, 0)
    gather_start(1, 1)

    def group_body(g, carry):
        for k in range(8):
            j = 8 * g + k
            k4 = k % 4
            gather_wait(k, k4)
            scatter_start(k, k4)

            if k < 4:
                idx_copies(j + 4, (k + 4) % 8)
            else:
                @pl.when(g < NGRP - 1)
                def _():
                    idx_copies(j + 4, (k + 4) % 8)

            if k < 2:
                @pl.when(g >= 1)
                def _():
                    scatter_wait((k - 2) % 8, (k - 2) % 4)
            else:
                scatter_wait(k - 2, (k - 2) % 4)

            if k < 6:
                idx_wait(j + 2, (k + 2) % 8)
                gather_start((k + 2) % 8, (k + 2) % 4)
            else:
                @pl.when(g < NGRP - 1)
                def _():
                    idx_wait(j + 2, (k + 2) % 8)
                    gather_start((k + 2) % 8, (k + 2) % 4)
        return carry

    lax.fori_loop(0, NGRP, group_body, 0)

    scatter_wait(6, 2)
    scatter_wait(7, 3)
    plsc.subcore_barrier()

    for b in range(ROW_BLOCKS):
        r = r0 + b * CHUNK
        pltpu.sync_copy(acc_sh.at[pl.ds(r, CHUNK)], rows[0])
        pltpu.sync_copy(rows[0], acc_out.at[c, pl.ds(r, CHUNK)])


CNT_BATCH = 8
CNT_CHUNK = 128
CNT_NCH = EPW // CNT_CHUNK
CNT_ROW_BLOCKS = ROWS_PER_TILE // CNT_CHUNK


@functools.partial(
    pl.kernel,
    out_type=jax.ShapeDtypeStruct((NUM_CORES, N_PAD, D), jnp.float32),
    scratch_types=[
        pltpu.VMEM_SHARED((N_PAD, D), jnp.float32),
        pltpu.VMEM((CNT_NCH, CNT_CHUNK), jnp.int32),
        pltpu.VMEM((CNT_CHUNK, D), jnp.float32),
        pltpu.SemaphoreType.DMA,
    ],
    mesh=_sc_mesh,
)
def _sc_count(dst_hbm, z_hbm, o_hbm, cnt_out, cnt_sh, didx_all, stage_v, csem):
    c = lax.axis_index("c")
    s = lax.axis_index("s")
    wid = c * NUM_SUBCORES + s
    r0 = s * ROWS_PER_TILE
    w0 = wid * CNT_NCH

    pltpu.sync_copy(dst_hbm.at[pl.ds(w0, CNT_NCH)], didx_all)
    pltpu.sync_copy(z_hbm, stage_v)
    for b in range(CNT_ROW_BLOCKS):
        pltpu.sync_copy(stage_v, cnt_sh.at[pl.ds(r0 + b * CNT_CHUNK,
                                                 CNT_CHUNK)])
    pltpu.sync_copy(o_hbm, stage_v)
    plsc.subcore_barrier()

    def batch_body(g, carry):
        for b in range(CNT_BATCH):
            pltpu.async_copy(
                stage_v, cnt_sh.at[didx_all.at[g * CNT_BATCH + b]], csem,
                add=True)
        for b in range(CNT_BATCH):
            pltpu.make_async_copy(
                stage_v, cnt_sh.at[didx_all.at[g * CNT_BATCH + b]],
                csem).wait()
        return carry

    lax.fori_loop(0, CNT_NCH // CNT_BATCH, batch_body, 0)
    plsc.subcore_barrier()

    for b in range(CNT_ROW_BLOCKS):
        r = r0 + b * CNT_CHUNK
        pltpu.sync_copy(cnt_sh.at[pl.ds(r, CNT_CHUNK)], stage_v)
        pltpu.sync_copy(stage_v, cnt_out.at[c, pl.ds(r, CNT_CHUNK)])


def _make_tc_layer(final: bool):
    BN = 1024

    def body(acc_ref, cnt_ref, x_ref, wl_ref, bl_ref, wr_ref, o_ref):
        a = acc_ref[0] + acc_ref[1]
        cnt = cnt_ref[0, :, 0:1] + cnt_ref[1, :, 0:1]
        agg = a * (1.0 / jnp.maximum(cnt, 1.0))
        hl = lax.dot_general(agg, wl_ref[...], (((1,), (1,)), ((), ())),
                             preferred_element_type=jnp.float32)
        hr = lax.dot_general(x_ref[...], wr_ref[...], (((1,), (1,)), ((), ())),
                             preferred_element_type=jnp.float32)
        o = hl + hr + bl_ref[...]
        if final:
            m = jnp.max(o, axis=1, keepdims=True)
            lse = jnp.log(jnp.sum(jnp.exp(o - m), axis=1, keepdims=True))
            o_ref[...] = o - m - lse
        else:
            o_ref[...] = jnp.tanh(o)

    grid = (N_PAD // BN,)
    return pl.pallas_call(
        body,
        grid=grid,
        in_specs=[
            pl.BlockSpec((NUM_CORES, BN, D), lambda i: (0, i, 0)),
            pl.BlockSpec((NUM_CORES, BN, D), lambda i: (0, i, 0)),
            pl.BlockSpec((BN, D), lambda i: (i, 0)),
            pl.BlockSpec((D, D), lambda i: (0, 0)),
            pl.BlockSpec((1, D), lambda i: (0, 0)),
            pl.BlockSpec((D, D), lambda i: (0, 0)),
        ],
        out_specs=pl.BlockSpec((BN, D), lambda i: (i, 0)),
        out_shape=jax.ShapeDtypeStruct((N_PAD, D), jnp.float32),
    )


_tc_hidden = _make_tc_layer(False)
_tc_final = _make_tc_layer(True)


def kernel(x, edge_index, W1l, b1l, W1r, W2l, b2l, W2r):
    src = edge_index[0]
    dst = edge_index[1]
    pad_e = E_PAD - E
    pad_i = jnp.arange(pad_e, dtype=jnp.int32)
    src_p = jnp.concatenate([src, pad_i % N])
    dst_p = jnp.concatenate([dst, N + pad_i % (N_PAD - N)])
    src2 = src_p.reshape(E_PAD // CHUNK, CHUNK)
    dst2 = dst_p.reshape(E_PAD // CHUNK, CHUNK)
    dst2c = dst_p.reshape(E_PAD // CNT_CHUNK, CNT_CHUNK)
    x_p = jnp.pad(x, ((0, N_PAD - N), (0, 0)))

    z = jnp.zeros((CHUNK, D), jnp.float32)
    zc = jnp.zeros((CNT_CHUNK, D), jnp.float32)
    oc = jnp.ones((CNT_CHUNK, D), jnp.float32)

    cnt = _sc_count(dst2c, zc, oc)
    acc1 = _sc_seg_sum(x_p, src2, dst2, z)
    h = _tc_hidden(acc1, cnt, x_p, W1l, b1l.reshape(1, D), W1r)
    acc2 = _sc_seg_sum(h, src2, dst2, z)
    o = _tc_final(acc2, cnt, h, W2l, b2l.reshape(1, D), W2r)
    return o[:N]
